# Initial kernel scaffold; baseline (speedup 1.0000x reference)
import jax
import jax.numpy as jnp
from jax.experimental import pallas as pl


def kernel(output, target):
    raise NotImplementedError("write your pallas kernel here")


# speedup vs baseline: 55.4550x; 55.4550x over previous
import jax
import jax.numpy as jnp
from jax import lax
from jax.experimental import pallas as pl
from jax.experimental.pallas import tpu as pltpu
from jax.experimental.pallas import tpu_sc as plsc

NUM_BINS = 256
ROWS = 64
COLS = 1048576
LANES = 16
NUM_CORES = 2
NUM_SUBCORES = 16
NUM_WORKERS = NUM_CORES * NUM_SUBCORES
ROWS_PER_WORKER = ROWS // NUM_WORKERS
CHUNK = 32768
NUM_CHUNKS = COLS // CHUNK
VECS_PER_CHUNK = CHUNK // LANES


def _hist_body(x_hbm, out_hbm, buf_a, buf_b, hist, hrow, sem_a, sem_b):
    wid = lax.axis_index("s") * NUM_CORES + lax.axis_index("c")
    lane_base = lax.iota(jnp.int32, LANES) * NUM_BINS
    ones = jnp.ones((LANES,), jnp.float32)
    zeros = jnp.zeros((LANES,), jnp.float32)
    bufs = (buf_a, buf_b)
    sems = (sem_a, sem_b)

    for r in range(ROWS_PER_WORKER):
        row = wid * ROWS_PER_WORKER + r

        def zero_body(j, carry):
            hist[pl.ds(j * LANES, LANES)] = zeros
            return carry

        lax.fori_loop(0, (LANES * NUM_BINS) // LANES, zero_body, 0)

        pending = pltpu.async_copy(
            x_hbm.at[row, pl.ds(0, CHUNK)], bufs[0], sems[0])
        for c in range(NUM_CHUNKS):
            if c + 1 < NUM_CHUNKS:
                nxt = pltpu.async_copy(
                    x_hbm.at[row, pl.ds((c + 1) * CHUNK, CHUNK)],
                    bufs[(c + 1) % 2], sems[(c + 1) % 2])
            pending.wait()
            buf = bufs[c % 2]

            def chunk_body(i, carry):
                v = buf[pl.ds(i * LANES, LANES)]
                b = (v * float(NUM_BINS)).astype(jnp.int32)
                b = jnp.minimum(jnp.maximum(b, 0), NUM_BINS - 1)
                plsc.addupdate_scatter(hist, [b + lane_base], ones)
                return carry

            lax.fori_loop(0, VECS_PER_CHUNK, chunk_body, 0)
            if c + 1 < NUM_CHUNKS:
                pending = nxt

        def red_body(j, carry):
            acc = hist[pl.ds(j * LANES, LANES)]
            for l in range(1, LANES):
                acc = acc + hist[pl.ds(l * NUM_BINS + j * LANES, LANES)]
            hrow[pl.ds(j * LANES, LANES)] = acc
            return carry

        lax.fori_loop(0, NUM_BINS // LANES, red_body, 0)
        pltpu.sync_copy(hrow, out_hbm.at[row])


_hist_kernel = pl.kernel(
    _hist_body,
    out_type=jax.ShapeDtypeStruct((ROWS, NUM_BINS), jnp.float32),
    mesh=plsc.VectorSubcoreMesh(
        core_axis_name="c", subcore_axis_name="s",
        num_cores=NUM_CORES, num_subcores=NUM_SUBCORES),
    compiler_params=pltpu.CompilerParams(needs_layout_passes=False),
    scratch_types=[
        pltpu.VMEM((CHUNK,), jnp.float32),
        pltpu.VMEM((CHUNK,), jnp.float32),
        pltpu.VMEM((LANES * NUM_BINS,), jnp.float32),
        pltpu.VMEM((NUM_BINS,), jnp.float32),
        pltpu.SemaphoreType.DMA,
        pltpu.SemaphoreType.DMA,
    ],
)


def _loss_body(counts_ref, target_ref, out_ref):
    counts = counts_ref[...]
    p = counts * (1.0 / COLS)
    logp = jnp.log(jnp.where(counts > 0.0, p, 1.0))
    ent = -jnp.sum(p * logp, axis=1, keepdims=True)
    d = ent - target_ref[...]
    out_ref[...] = jnp.reshape(jnp.sum(d * d) * (1.0 / ROWS), (1, 1))


def kernel(output, target):
    counts = _hist_kernel(output)
    loss = pl.pallas_call(
        _loss_body,
        out_shape=jax.ShapeDtypeStruct((1, 1), jnp.float32),
    )(counts, target.reshape(ROWS, 1))
    return loss[0, 0]


# speedup vs baseline: 317.4897x; 5.7252x over previous
import jax
import jax.numpy as jnp
from jax import lax
from jax.experimental import pallas as pl
from jax.experimental.pallas import tpu as pltpu
from jax.experimental.pallas import tpu_sc as plsc

NUM_BINS = 256
ROWS = 64
COLS = 1048576
LANES = 16
NUM_CORES = 2
NUM_SUBCORES = 16
NUM_WORKERS = NUM_CORES * NUM_SUBCORES
ROWS_PER_WORKER = ROWS // NUM_WORKERS
CHUNK = 32768
NUM_CHUNKS = COLS // CHUNK
VECS_PER_CHUNK = CHUNK // LANES


def _hist_body(x_hbm, out_hbm, buf_a, buf_b, hist, hrow, sem_a, sem_b):
    wid = lax.axis_index("s") * NUM_CORES + lax.axis_index("c")
    lane_base = lax.iota(jnp.int32, LANES) * NUM_BINS
    ones = jnp.ones((LANES,), jnp.float32)
    zeros = jnp.zeros((LANES,), jnp.float32)

    def process(buf):
        @plsc.parallel_loop(0, CHUNK, step=LANES, unroll=8)
        def _(i):
            v = buf[pl.ds(i, LANES)]
            b = (v * float(NUM_BINS)).astype(jnp.int32)
            plsc.addupdate_scatter(hist, [b + lane_base], ones)

    for r in range(ROWS_PER_WORKER):
        row = wid * ROWS_PER_WORKER + r

        @plsc.parallel_loop(0, LANES * NUM_BINS, step=LANES)
        def _(j):
            hist[pl.ds(j, LANES)] = zeros

        pltpu.async_copy(x_hbm.at[row, pl.ds(0, CHUNK)], buf_a, sem_a)

        def outer(k, carry):
            base = k * (2 * CHUNK)
            pltpu.async_copy(
                x_hbm.at[row, pl.ds(base + CHUNK, CHUNK)], buf_b, sem_b)
            pltpu.make_async_copy(
                x_hbm.at[row, pl.ds(base, CHUNK)], buf_a, sem_a).wait()
            process(buf_a)

            @pl.when(k < NUM_CHUNKS // 2 - 1)
            def _():
                pltpu.async_copy(
                    x_hbm.at[row, pl.ds(base + 2 * CHUNK, CHUNK)],
                    buf_a, sem_a)

            pltpu.make_async_copy(
                x_hbm.at[row, pl.ds(base + CHUNK, CHUNK)], buf_b, sem_b).wait()
            process(buf_b)
            return carry

        lax.fori_loop(0, NUM_CHUNKS // 2, outer, 0)

        @plsc.parallel_loop(0, NUM_BINS, step=LANES)
        def _(j):
            acc = hist[pl.ds(j, LANES)]
            for l in range(1, LANES):
                acc = acc + hist[pl.ds(l * NUM_BINS + j, LANES)]
            hrow[pl.ds(j, LANES)] = acc

        pltpu.sync_copy(hrow, out_hbm.at[row])


_hist_kernel = pl.kernel(
    _hist_body,
    out_type=jax.ShapeDtypeStruct((ROWS, NUM_BINS), jnp.float32),
    mesh=plsc.VectorSubcoreMesh(
        core_axis_name="c", subcore_axis_name="s",
        num_cores=NUM_CORES, num_subcores=NUM_SUBCORES),
    compiler_params=pltpu.CompilerParams(needs_layout_passes=False),
    scratch_types=[
        pltpu.VMEM((CHUNK,), jnp.float32),
        pltpu.VMEM((CHUNK,), jnp.float32),
        pltpu.VMEM((LANES * NUM_BINS,), jnp.float32),
        pltpu.VMEM((NUM_BINS,), jnp.float32),
        pltpu.SemaphoreType.DMA,
        pltpu.SemaphoreType.DMA,
    ],
)


def _loss_body(counts_ref, target_ref, out_ref):
    counts = counts_ref[...]
    p = counts * (1.0 / COLS)
    logp = jnp.log(jnp.where(counts > 0.0, p, 1.0))
    ent = -jnp.sum(p * logp, axis=1, keepdims=True)
    d = ent - target_ref[...]
    out_ref[...] = jnp.reshape(jnp.sum(d * d) * (1.0 / ROWS), (1, 1))


def kernel(output, target):
    counts = _hist_kernel(output)
    loss = pl.pallas_call(
        _loss_body,
        out_shape=jax.ShapeDtypeStruct((1, 1), jnp.float32),
    )(counts, target.reshape(ROWS, 1))
    return loss[0, 0]


# speedup vs baseline: 318.8190x; 1.0042x over previous
import jax
import jax.numpy as jnp
from jax import lax
from jax.experimental import pallas as pl
from jax.experimental.pallas import tpu as pltpu
from jax.experimental.pallas import tpu_sc as plsc

NUM_BINS = 256
ROWS = 64
COLS = 1048576
LANES = 16
NUM_CORES = 2
NUM_SUBCORES = 16
NUM_WORKERS = NUM_CORES * NUM_SUBCORES
ROWS_PER_WORKER = ROWS // NUM_WORKERS
CHUNK = 32768
NUM_CHUNKS = COLS // CHUNK
VECS_PER_CHUNK = CHUNK // LANES


def _hist_body(x_hbm, out_hbm, buf_a, buf_b, hist, hrow, sem_a, sem_b):
    wid = lax.axis_index("s") * NUM_CORES + lax.axis_index("c")
    lane_base = lax.iota(jnp.int32, LANES) * NUM_BINS
    ones = jnp.ones((LANES,), jnp.float32)
    zeros = jnp.zeros((LANES,), jnp.float32)

    def process(buf):
        @plsc.parallel_loop(0, CHUNK, step=LANES, unroll=16)
        def _(i):
            v = buf[pl.ds(i, LANES)]
            b = (v * float(NUM_BINS)).astype(jnp.int32)
            plsc.addupdate_scatter(hist, [b + lane_base], ones)

    for r in range(ROWS_PER_WORKER):
        row = wid * ROWS_PER_WORKER + r

        @plsc.parallel_loop(0, LANES * NUM_BINS, step=LANES)
        def _(j):
            hist[pl.ds(j, LANES)] = zeros

        pltpu.async_copy(x_hbm.at[row, pl.ds(0, CHUNK)], buf_a, sem_a)

        def outer(k, carry):
            base = k * (2 * CHUNK)
            pltpu.async_copy(
                x_hbm.at[row, pl.ds(base + CHUNK, CHUNK)], buf_b, sem_b)
            pltpu.make_async_copy(
                x_hbm.at[row, pl.ds(base, CHUNK)], buf_a, sem_a).wait()
            process(buf_a)

            @pl.when(k < NUM_CHUNKS // 2 - 1)
            def _():
                pltpu.async_copy(
                    x_hbm.at[row, pl.ds(base + 2 * CHUNK, CHUNK)],
                    buf_a, sem_a)

            pltpu.make_async_copy(
                x_hbm.at[row, pl.ds(base + CHUNK, CHUNK)], buf_b, sem_b).wait()
            process(buf_b)
            return carry

        lax.fori_loop(0, NUM_CHUNKS // 2, outer, 0)

        @plsc.parallel_loop(0, NUM_BINS, step=LANES)
        def _(j):
            acc = hist[pl.ds(j, LANES)]
            for l in range(1, LANES):
                acc = acc + hist[pl.ds(l * NUM_BINS + j, LANES)]
            hrow[pl.ds(j, LANES)] = acc

        pltpu.sync_copy(hrow, out_hbm.at[row])


_hist_kernel = pl.kernel(
    _hist_body,
    out_type=jax.ShapeDtypeStruct((ROWS, NUM_BINS), jnp.float32),
    mesh=plsc.VectorSubcoreMesh(
        core_axis_name="c", subcore_axis_name="s",
        num_cores=NUM_CORES, num_subcores=NUM_SUBCORES),
    compiler_params=pltpu.CompilerParams(needs_layout_passes=False),
    scratch_types=[
        pltpu.VMEM((CHUNK,), jnp.float32),
        pltpu.VMEM((CHUNK,), jnp.float32),
        pltpu.VMEM((LANES * NUM_BINS,), jnp.float32),
        pltpu.VMEM((NUM_BINS,), jnp.float32),
        pltpu.SemaphoreType.DMA,
        pltpu.SemaphoreType.DMA,
    ],
)


def _loss_body(counts_ref, target_ref, out_ref):
    counts = counts_ref[...]
    p = counts * (1.0 / COLS)
    logp = jnp.log(jnp.where(counts > 0.0, p, 1.0))
    ent = -jnp.sum(p * logp, axis=1, keepdims=True)
    d = ent - target_ref[...]
    out_ref[...] = jnp.reshape(jnp.sum(d * d) * (1.0 / ROWS), (1, 1))


def kernel(output, target):
    counts = _hist_kernel(output)
    loss = pl.pallas_call(
        _loss_body,
        out_shape=jax.ShapeDtypeStruct((1, 1), jnp.float32),
    )(counts, target.reshape(ROWS, 1))
    return loss[0, 0]
